# baseline (speedup 1.0000x reference)
import functools

import jax
import jax.numpy as jnp
from jax import lax
from jax.experimental import pallas as pl
from jax.experimental.pallas import tpu as pltpu
from jax.experimental.pallas import tpu_sc as plsc

VOCAB = 1000000
EMBED_DIM = 64
NUM_CLASS = 4
L = 16384

NC = 2
NS = 16
NW = NC * NS
N_CHUNKS = 4
CHUNK = L // NW // N_CHUNKS
VOCAB_PAD = 1000064
VSLICE = VOCAB_PAD // NS
ZCHUNK = 500

_mesh = plsc.VectorSubcoreMesh(
    core_axis_name="c", subcore_axis_name="s", num_cores=NC, num_subcores=NS
)


@functools.partial(
    pl.kernel,
    out_type=[
        jax.ShapeDtypeStruct((VOCAB_PAD,), jnp.float32),
        jax.ShapeDtypeStruct((VOCAB_PAD,), jnp.float32),
    ],
    mesh=_mesh,
    scratch_types=[
        pltpu.VMEM((N_CHUNKS, CHUNK), jnp.int32),
        pltpu.VMEM((CHUNK,), jnp.float32),
        pltpu.VMEM((ZCHUNK * 16,), jnp.float32),
        pltpu.VMEM((2 * ZCHUNK * 16,), jnp.float32),
        pltpu.VMEM_SHARED((VOCAB_PAD,), jnp.float32),
        pltpu.SemaphoreType.DMA,
    ],
)
def _sc_counts(idx_hbm, out0_hbm, out1_hbm, idx_v, ones_v, zbuf_v, ebuf_v, counts_sh, sem):
    cid = lax.axis_index("c")
    sid = lax.axis_index("s")
    wid = sid * NC + cid

    icopies = [
        pltpu.async_copy(
            idx_hbm.at[pl.ds(wid * (N_CHUNKS * CHUNK) + j * CHUNK, CHUNK)],
            idx_v.at[j],
            sem,
        )
        for j in range(N_CHUNKS)
    ]
    for c in icopies:
        c.wait()
    for k in range(CHUNK // 16):
        ones_v[pl.ds(k * 16, 16)] = jnp.ones((16,), jnp.float32)

    def zbody(k, _):
        zbuf_v[pl.ds(k * 16, 16)] = jnp.zeros((16,), jnp.float32)
        return 0

    lax.fori_loop(0, ZCHUNK, zbody, 0)
    base = sid * VSLICE
    zcopies = [
        pltpu.async_copy(
            zbuf_v, counts_sh.at[pl.ds(base + k * ZCHUNK * 16, ZCHUNK * 16)], sem
        )
        for k in range(VSLICE // (ZCHUNK * 16))
    ]
    rem = VSLICE % (ZCHUNK * 16)
    if rem:
        zcopies.append(
            pltpu.async_copy(
                zbuf_v.at[pl.ds(0, rem)],
                counts_sh.at[pl.ds(base + VSLICE - rem, rem)],
                sem,
            )
        )
    for c in zcopies:
        c.wait()
    plsc.subcore_barrier()

    for j in range(N_CHUNKS):
        pltpu.sync_copy(ones_v, counts_sh.at[idx_v.at[j]], add=True)
    plsc.subcore_barrier()

    EC = ZCHUNK * 16
    n_full = VSLICE // EC
    rem = VSLICE % EC
    sizes = [EC] * n_full + ([rem] if rem else [])

    def export_to(out_hbm):
        pend = {}
        for k, n in enumerate(sizes):
            h = (k % 2) * EC
            if k - 2 in pend:
                pend.pop(k - 2).wait()
            off = base + k * EC
            pltpu.sync_copy(counts_sh.at[pl.ds(off, n)], ebuf_v.at[pl.ds(h, n)])
            pend[k] = pltpu.async_copy(
                ebuf_v.at[pl.ds(h, n)], out_hbm.at[pl.ds(off, n)], sem
            )
        for c in pend.values():
            c.wait()

    @pl.when(cid == 0)
    def _():
        export_to(out0_hbm)

    @pl.when(cid == 1)
    def _():
        export_to(out1_hbm)


SWEEP_BLK = 32768
SC_COLS_PER_PAIR = 26624
SPLIT = 16 * SC_COLS_PER_PAIR
SC_CHUNK = 1024
SC_N_CHUNKS = SC_COLS_PER_PAIR // SC_CHUNK
FEATS_PER_TILE = 32
TC_STEPS = (VOCAB - SPLIT + SWEEP_BLK - 1) // SWEEP_BLK
TC_BLK0 = SPLIT // SWEEP_BLK


def _tc_sweep(tt_ref, c0_ref, c1_ref, o_ref, acc_ref):
    pid = pl.program_id(0)

    @pl.when(pid == 0)
    def _():
        acc_ref[...] = jnp.zeros_like(acc_ref)

    cb = c0_ref[...] + c1_ref[...]
    prod = tt_ref[...] * cb[None, :]
    cols = (pid + TC_BLK0) * SWEEP_BLK + lax.broadcasted_iota(
        jnp.int32, (1, SWEEP_BLK), 1
    )
    prod = jnp.where(cols < VOCAB, prod, 0.0)
    acc_ref[...] += jnp.sum(prod, axis=1).reshape(1, EMBED_DIM)

    @pl.when(pid == TC_STEPS - 1)
    def _():
        o_ref[...] = acc_ref[...]


@functools.partial(
    pl.kernel,
    out_type=jax.ShapeDtypeStruct((NW * FEATS_PER_TILE * 16,), jnp.float32),
    mesh=_mesh,
    scratch_types=[
        pltpu.VMEM((3, FEATS_PER_TILE, SC_CHUNK), jnp.float32),
        pltpu.VMEM((3 * SC_CHUNK,), jnp.float32),
        pltpu.VMEM((3 * SC_CHUNK,), jnp.float32),
        pltpu.VMEM((FEATS_PER_TILE * 16,), jnp.float32),
        pltpu.SemaphoreType.DMA,
    ],
)
def _sc_sweep(tt_hbm, c0_hbm, c1_hbm, out_hbm, tbuf, c0b, c1b, part_v, sem):
    cid = lax.axis_index("c")
    sid = lax.axis_index("s")
    wid = sid * NC + cid
    pair = wid // 2
    fbase = (wid % 2) * FEATS_PER_TILE
    vbase = pair * SC_COLS_PER_PAIR

    def issue(k):
        b = k % 3
        off = vbase + k * SC_CHUNK
        return [
            pltpu.async_copy(
                tt_hbm.at[pl.ds(fbase, FEATS_PER_TILE), pl.ds(off, SC_CHUNK)],
                tbuf.at[b],
                sem,
            ),
            pltpu.async_copy(
                c0_hbm.at[pl.ds(off, SC_CHUNK)], c0b.at[pl.ds(b * SC_CHUNK, SC_CHUNK)], sem
            ),
            pltpu.async_copy(
                c1_hbm.at[pl.ds(off, SC_CHUNK)], c1b.at[pl.ds(b * SC_CHUNK, SC_CHUNK)], sem
            ),
        ]

    zeros = jnp.zeros((16,), jnp.float32)
    accs = [zeros] * FEATS_PER_TILE
    pend = {0: issue(0), 1: issue(1)}
    for k in range(SC_N_CHUNKS):
        for c in pend.pop(k):
            c.wait()
        if k + 2 < SC_N_CHUNKS:
            pend[k + 2] = issue(k + 2)
        b = k % 3

        def body(g, acc_t):
            cnt = (
                c0b[pl.ds(b * SC_CHUNK + g * 16, 16)]
                + c1b[pl.ds(b * SC_CHUNK + g * 16, 16)]
            )
            return tuple(
                acc_t[e] + tbuf[b, e, pl.ds(g * 16, 16)] * cnt
                for e in range(FEATS_PER_TILE)
            )

        accs = list(lax.fori_loop(0, SC_CHUNK // 16, body, tuple(accs)))

    for e in range(FEATS_PER_TILE):
        part_v[pl.ds(e * 16, 16)] = accs[e]
    outc = [
        pltpu.async_copy(
            part_v.at[pl.ds(e * 16, 16)],
            out_hbm.at[pl.ds((fbase + e) * (16 * NS) + pair * 16, 16)],
            sem,
        )
        for e in range(FEATS_PER_TILE)
    ]
    for c in outc:
        c.wait()


def _tc_head(tc_ref, sc_ref, w_ref, b_ref, o_ref):
    sc_sum = jnp.sum(sc_ref[...], axis=1).reshape(1, EMBED_DIM)
    emb = (tc_ref[...] + sc_sum) * (1.0 / L)
    o_ref[...] = (
        jnp.dot(emb, w_ref[...].T, preferred_element_type=jnp.float32) + b_ref[...]
    )


def kernel(text, emb_table, fc_w, fc_b):
    c0, c1 = _sc_counts(text.astype(jnp.int32))
    tt = emb_table.T
    tc_part = pl.pallas_call(
        _tc_sweep,
        grid=(TC_STEPS,),
        in_specs=[
            pl.BlockSpec((EMBED_DIM, SWEEP_BLK), lambda i: (0, i + TC_BLK0)),
            pl.BlockSpec((SWEEP_BLK,), lambda i: (i + TC_BLK0,)),
            pl.BlockSpec((SWEEP_BLK,), lambda i: (i + TC_BLK0,)),
        ],
        out_specs=pl.BlockSpec((1, EMBED_DIM), lambda i: (0, 0)),
        out_shape=jax.ShapeDtypeStruct((1, EMBED_DIM), jnp.float32),
        scratch_shapes=[pltpu.VMEM((1, EMBED_DIM), jnp.float32)],
    )(tt, c0, c1)
    sc_parts = _sc_sweep(tt, c0, c1)
    out = pl.pallas_call(
        _tc_head,
        out_shape=jax.ShapeDtypeStruct((1, NUM_CLASS), jnp.float32),
    )(
        tc_part,
        sc_parts.reshape(EMBED_DIM, 16 * NS),
        fc_w,
        fc_b.reshape(1, NUM_CLASS),
    )
    return out


# speedup vs baseline: 1.0659x; 1.0659x over previous
import functools

import jax
import jax.numpy as jnp
from jax import lax
from jax.experimental import pallas as pl
from jax.experimental.pallas import tpu as pltpu
from jax.experimental.pallas import tpu_sc as plsc

VOCAB = 1000000
EMBED_DIM = 64
NUM_CLASS = 4
L = 16384

NC = 2
NS = 16
NW = NC * NS
N_CHUNKS = 4
CHUNK = L // NW // N_CHUNKS
VOCAB_PAD = 1000064
VSLICE = VOCAB_PAD // NS
ZCHUNK = 500

_mesh = plsc.VectorSubcoreMesh(
    core_axis_name="c", subcore_axis_name="s", num_cores=NC, num_subcores=NS
)


@functools.partial(
    pl.kernel,
    out_type=[
        jax.ShapeDtypeStruct((VOCAB_PAD,), jnp.float32),
        jax.ShapeDtypeStruct((VOCAB_PAD,), jnp.float32),
    ],
    mesh=_mesh,
    scratch_types=[
        pltpu.VMEM((N_CHUNKS, CHUNK), jnp.int32),
        pltpu.VMEM((CHUNK,), jnp.float32),
        pltpu.VMEM((ZCHUNK * 16,), jnp.float32),
        pltpu.VMEM((2 * ZCHUNK * 16,), jnp.float32),
        pltpu.VMEM_SHARED((VOCAB_PAD,), jnp.float32),
        pltpu.SemaphoreType.DMA,
    ],
)
def _sc_counts(idx_hbm, out0_hbm, out1_hbm, idx_v, ones_v, zbuf_v, ebuf_v, counts_sh, sem):
    cid = lax.axis_index("c")
    sid = lax.axis_index("s")
    wid = sid * NC + cid

    icopies = [
        pltpu.async_copy(
            idx_hbm.at[pl.ds(wid * (N_CHUNKS * CHUNK) + j * CHUNK, CHUNK)],
            idx_v.at[j],
            sem,
        )
        for j in range(N_CHUNKS)
    ]
    for c in icopies:
        c.wait()
    for k in range(CHUNK // 16):
        ones_v[pl.ds(k * 16, 16)] = jnp.ones((16,), jnp.float32)

    def zbody(k, _):
        zbuf_v[pl.ds(k * 16, 16)] = jnp.zeros((16,), jnp.float32)
        return 0

    lax.fori_loop(0, ZCHUNK, zbody, 0)
    base = sid * VSLICE
    zcopies = [
        pltpu.async_copy(
            zbuf_v, counts_sh.at[pl.ds(base + k * ZCHUNK * 16, ZCHUNK * 16)], sem
        )
        for k in range(VSLICE // (ZCHUNK * 16))
    ]
    rem = VSLICE % (ZCHUNK * 16)
    if rem:
        zcopies.append(
            pltpu.async_copy(
                zbuf_v.at[pl.ds(0, rem)],
                counts_sh.at[pl.ds(base + VSLICE - rem, rem)],
                sem,
            )
        )
    for c in zcopies:
        c.wait()
    plsc.subcore_barrier()

    for j in range(N_CHUNKS):
        pltpu.sync_copy(ones_v, counts_sh.at[idx_v.at[j]], add=True)
    plsc.subcore_barrier()

    EC = ZCHUNK * 16
    n_full = VSLICE // EC
    rem = VSLICE % EC
    sizes = [EC] * n_full + ([rem] if rem else [])

    def export_to(out_hbm):
        pend = {}
        for k, n in enumerate(sizes):
            h = (k % 2) * EC
            if k - 2 in pend:
                pend.pop(k - 2).wait()
            off = base + k * EC
            pltpu.sync_copy(counts_sh.at[pl.ds(off, n)], ebuf_v.at[pl.ds(h, n)])
            pend[k] = pltpu.async_copy(
                ebuf_v.at[pl.ds(h, n)], out_hbm.at[pl.ds(off, n)], sem
            )
        for c in pend.values():
            c.wait()

    @pl.when(cid == 0)
    def _():
        export_to(out0_hbm)

    @pl.when(cid == 1)
    def _():
        export_to(out1_hbm)


SWEEP_BLK = 65536
TC_STEPS = (VOCAB + SWEEP_BLK - 1) // SWEEP_BLK


def _tc_sweep(tt_ref, c0_ref, c1_ref, w_ref, b_ref, o_ref, acc_ref):
    pid = pl.program_id(0)

    @pl.when(pid == 0)
    def _():
        acc_ref[...] = jnp.zeros_like(acc_ref)

    cb = c0_ref[...] + c1_ref[...]
    prod = tt_ref[...] * cb[None, :]
    cols = pid * SWEEP_BLK + lax.broadcasted_iota(jnp.int32, (1, SWEEP_BLK), 1)
    prod = jnp.where(cols < VOCAB, prod, 0.0)
    acc_ref[...] += jnp.sum(prod, axis=1).reshape(1, EMBED_DIM)

    @pl.when(pid == TC_STEPS - 1)
    def _():
        emb = acc_ref[...] * (1.0 / L)
        o_ref[...] = (
            jnp.dot(emb, w_ref[...].T, preferred_element_type=jnp.float32)
            + b_ref[...]
        )


def kernel(text, emb_table, fc_w, fc_b):
    c0, c1 = _sc_counts(text.astype(jnp.int32))
    tt = emb_table.T
    out = pl.pallas_call(
        _tc_sweep,
        grid=(TC_STEPS,),
        in_specs=[
            pl.BlockSpec((EMBED_DIM, SWEEP_BLK), lambda i: (0, i)),
            pl.BlockSpec((SWEEP_BLK,), lambda i: (i,)),
            pl.BlockSpec((SWEEP_BLK,), lambda i: (i,)),
            pl.BlockSpec((NUM_CLASS, EMBED_DIM), lambda i: (0, 0)),
            pl.BlockSpec((1, NUM_CLASS), lambda i: (0, 0)),
        ],
        out_specs=pl.BlockSpec((1, NUM_CLASS), lambda i: (0, 0)),
        out_shape=jax.ShapeDtypeStruct((1, NUM_CLASS), jnp.float32),
        scratch_shapes=[pltpu.VMEM((1, EMBED_DIM), jnp.float32)],
    )(tt, c0, c1, fc_w, fc_b.reshape(1, NUM_CLASS))
    return out
